# baseline (speedup 1.0000x reference)
import jax
import jax.numpy as jnp
from jax import lax
from jax.experimental import pallas as pl
from jax.experimental.pallas import tpu as pltpu
from jax.experimental.pallas import tpu_sc as plsc

EMB = 64
WINDOW = 200
PATH = 20
NC = 2
NS = 16
L = 16
WPT = 8
NTILES_W = WINDOW // WPT
THETA_TILE = NTILES_W


def _sc_body(u_hbm, v_hbm, words_hbm, path_hbm, partials_hbm, theta_hbm,
             widx_v, wrows_v, acc_v, pidx_v, prows_v, sem):
    wid = lax.axis_index("s") * NC + lax.axis_index("c")

    @pl.when(wid < NTILES_W)
    def _():
        base = pl.multiple_of(wid * WPT, WPT)
        pltpu.sync_copy(words_hbm.at[pl.ds(base, WPT)], widx_v)
        pltpu.async_copy(u_hbm.at[widx_v], wrows_v, sem).wait()
        for c in range(EMB // L):
            a = wrows_v[0, pl.ds(c * L, L)]
            for r in range(1, WPT):
                a = a + wrows_v[r, pl.ds(c * L, L)]
            acc_v[pl.ds(c * L, L)] = a
        pltpu.sync_copy(acc_v, partials_hbm.at[wid])

    @pl.when(wid == THETA_TILE)
    def _():
        pltpu.sync_copy(path_hbm, pidx_v)
        pltpu.async_copy(v_hbm.at[pidx_v], prows_v, sem).wait()
        pltpu.sync_copy(prows_v, theta_hbm)


def _sc_gather(words, h_path, u_emb, v_emb):
    mesh = plsc.VectorSubcoreMesh(core_axis_name="c", subcore_axis_name="s")
    f = pl.kernel(
        _sc_body,
        out_type=(
            jax.ShapeDtypeStruct((NTILES_W, EMB), jnp.float32),
            jax.ShapeDtypeStruct((PATH, EMB), jnp.float32),
        ),
        mesh=mesh,
        scratch_types=[
            pltpu.VMEM((WPT,), jnp.int32),
            pltpu.VMEM((WPT, EMB), jnp.float32),
            pltpu.VMEM((EMB,), jnp.float32),
            pltpu.VMEM((PATH,), jnp.int32),
            pltpu.VMEM((PATH, EMB), jnp.float32),
            pltpu.SemaphoreType.DMA,
        ],
        compiler_params=pltpu.CompilerParams(use_tc_tiling_on_sc=False),
    )
    return f(u_emb, v_emb, words, h_path)


def _tc_body(partials_ref, theta_ref, hcode_ref, out_ref):
    xw = jnp.sum(partials_ref[...], axis=0, keepdims=True)
    t = jnp.sum(theta_ref[...] * xw, axis=1, keepdims=True)
    z = jax.nn.sigmoid(t)
    hc = hcode_ref[...]
    loss = jnp.log(z) * hc + jnp.log(1.0 - z) * (1.0 - hc)
    out_ref[...] = jnp.sum(loss, axis=0, keepdims=True)


def _tc_finish(partials, theta, h_code):
    return pl.pallas_call(
        _tc_body,
        out_shape=jax.ShapeDtypeStruct((1, 1), jnp.float32),
    )(partials, theta, h_code.reshape(PATH, 1))


def kernel(words, h_code, h_path, u_emb, v_emb):
    words = words.astype(jnp.int32)
    h_path = h_path.astype(jnp.int32)
    partials, theta = _sc_gather(words, h_path, u_emb, v_emb)
    return _tc_finish(partials, theta, h_code)


# speedup vs baseline: 1.5875x; 1.5875x over previous
import jax
import jax.numpy as jnp
from jax import lax
from jax.experimental import pallas as pl
from jax.experimental.pallas import tpu as pltpu
from jax.experimental.pallas import tpu_sc as plsc

EMB = 64
WINDOW = 200
PATH = 20
NC = 2
NS = 16
L = 16
WPT = 8
NTILES_W = WINDOW // WPT
THETA_TILE = NTILES_W


def _sc_body(u_hbm, v_hbm, words_hbm, path_hbm, partials_hbm, theta_hbm,
             widx_v, wrows_v, acc_v, pidx_v, prows_v, sem):
    wid = lax.axis_index("s") * NC + lax.axis_index("c")

    @pl.when(wid < NTILES_W)
    def _():
        base = pl.multiple_of(wid * WPT, WPT)
        pltpu.sync_copy(words_hbm.at[pl.ds(base, WPT)], widx_v.at[pl.ds(0, WPT)])
        wvec = widx_v[...]
        for r in range(WPT):
            idx = wvec[r]
            pltpu.async_copy(u_hbm.at[pl.ds(idx, 1)], wrows_v.at[pl.ds(r, 1)], sem)
        for r in range(WPT):
            pltpu.make_async_copy(u_hbm.at[pl.ds(0, 1)], wrows_v.at[pl.ds(r, 1)], sem).wait()
        for c in range(EMB // L):
            a = wrows_v[0, pl.ds(c * L, L)]
            for r in range(1, WPT):
                a = a + wrows_v[r, pl.ds(c * L, L)]
            acc_v[pl.ds(c * L, L)] = a
        pltpu.sync_copy(acc_v, partials_hbm.at[wid])

    @pl.when(wid == THETA_TILE)
    def _():
        pltpu.sync_copy(path_hbm, pidx_v.at[pl.ds(0, PATH)])
        pvec0 = pidx_v[pl.ds(0, L)]
        pvec1 = pidx_v[pl.ds(L, L)]
        for r in range(PATH):
            idx = pvec0[r] if r < L else pvec1[r - L]
            pltpu.async_copy(v_hbm.at[pl.ds(idx, 1)], prows_v.at[pl.ds(r, 1)], sem)
        for r in range(PATH):
            pltpu.make_async_copy(v_hbm.at[pl.ds(0, 1)], prows_v.at[pl.ds(r, 1)], sem).wait()
        pltpu.sync_copy(prows_v, theta_hbm)


def _sc_gather(words, h_path, u_emb, v_emb):
    mesh = plsc.VectorSubcoreMesh(core_axis_name="c", subcore_axis_name="s")
    f = pl.kernel(
        _sc_body,
        out_type=(
            jax.ShapeDtypeStruct((NTILES_W, EMB), jnp.float32),
            jax.ShapeDtypeStruct((PATH, EMB), jnp.float32),
        ),
        mesh=mesh,
        scratch_types=[
            pltpu.VMEM((L,), jnp.int32),
            pltpu.VMEM((WPT, EMB), jnp.float32),
            pltpu.VMEM((EMB,), jnp.float32),
            pltpu.VMEM((2 * L,), jnp.int32),
            pltpu.VMEM((PATH, EMB), jnp.float32),
            pltpu.SemaphoreType.DMA,
        ],
    )
    return f(u_emb, v_emb, words, h_path)


def _tc_body(partials_ref, theta_ref, hcode_ref, out_ref):
    xw = jnp.sum(partials_ref[...], axis=0, keepdims=True)
    t = jnp.sum(theta_ref[...] * xw, axis=1, keepdims=True)
    z = jax.nn.sigmoid(t)
    hc = hcode_ref[...]
    loss = jnp.log(z) * hc + jnp.log(1.0 - z) * (1.0 - hc)
    out_ref[...] = jnp.sum(loss, axis=0, keepdims=True)


def _tc_finish(partials, theta, h_code):
    return pl.pallas_call(
        _tc_body,
        out_shape=jax.ShapeDtypeStruct((1, 1), jnp.float32),
    )(partials, theta, h_code.reshape(PATH, 1))


def kernel(words, h_code, h_path, u_emb, v_emb):
    words = words.astype(jnp.int32)
    h_path = h_path.astype(jnp.int32)
    partials, theta = _sc_gather(words, h_path, u_emb, v_emb)
    return _tc_finish(partials, theta, h_code)


# speedup vs baseline: 44.6110x; 28.1019x over previous
import jax
import jax.numpy as jnp
from jax import lax
from jax.experimental import pallas as pl
from jax.experimental.pallas import tpu as pltpu
from jax.experimental.pallas import tpu_sc as plsc

EMB = 64
WINDOW = 200
PATH = 20
PATH_PAD = 24
NC = 2
NS = 16
L = 16
LANES = 128
WPT = 8
NTILES_W = WINDOW // WPT
NTILES_P = PATH_PAD // WPT


def _sc_body(ut_hbm, vt_hbm, words_hbm, path_hbm, partials_hbm, theta_hbm,
             idx_v, buf_v, acc_v, sem):
    wid = lax.axis_index("s") * NC + lax.axis_index("c")
    rowi = lax.iota(jnp.int32, L)

    def load_idx(src, base):
        pltpu.sync_copy(src.at[pl.ds(base, WPT)], idx_v.at[pl.ds(0, WPT)])
        ivec = idx_v[...]
        return (ivec // LANES) * LANES, ivec % LANES

    def fire(tab, tcol):
        def body(r, carry):
            rvec = jnp.broadcast_to(r, (L,))
            cb = pl.multiple_of(tcol.at[rvec].get(mode="promise_in_bounds")[0],
                                LANES)
            pltpu.async_copy(tab.at[:, pl.ds(cb, LANES)], buf_v.at[r], sem)
            return carry
        lax.fori_loop(0, WPT, body, 0)

    def extract(lane, r):
        rvec = jnp.broadcast_to(r, (L,))
        laneb = lane.at[rvec].get(mode="promise_in_bounds")
        return [plsc.load_gather(buf_v, [rvec, rowi + (c * L), laneb])
                for c in range(EMB // L)]

    @pl.when(wid < NTILES_W)
    def _():
        tcol, lane = load_idx(words_hbm, pl.multiple_of(wid * WPT, WPT))
        fire(ut_hbm, tcol)

        def drain(r, accs):
            pltpu.make_async_copy(ut_hbm.at[:, pl.ds(0, LANES)], buf_v.at[r],
                                  sem).wait()
            g = extract(lane, r)
            return tuple(a + b for a, b in zip(accs, g))

        accs = lax.fori_loop(0, WPT, drain,
                             tuple(jnp.zeros((L,), jnp.float32)
                                   for _ in range(EMB // L)))
        for c in range(EMB // L):
            acc_v[pl.ds(c * L, L)] = accs[c]
        pltpu.sync_copy(acc_v, partials_hbm.at[wid])

    @pl.when(jnp.logical_and(wid >= NTILES_W, wid < NTILES_W + NTILES_P))
    def _():
        pbase = pl.multiple_of((wid - NTILES_W) * WPT, WPT)
        tcol, lane = load_idx(path_hbm, pbase)
        fire(vt_hbm, tcol)

        def drain(r, carry):
            pltpu.make_async_copy(vt_hbm.at[:, pl.ds(0, LANES)], buf_v.at[r],
                                  sem).wait()
            g = extract(lane, r)
            for c in range(EMB // L):
                acc_v[pl.ds(c * L, L)] = g[c]
            pltpu.sync_copy(acc_v, theta_hbm.at[pbase + r])
            return carry

        lax.fori_loop(0, WPT, drain, 0)


def _sc_gather(words, h_path_pad, ut, vt):
    mesh = plsc.VectorSubcoreMesh(core_axis_name="c", subcore_axis_name="s")
    f = pl.kernel(
        _sc_body,
        out_type=(
            jax.ShapeDtypeStruct((NTILES_W, EMB), jnp.float32),
            jax.ShapeDtypeStruct((PATH_PAD, EMB), jnp.float32),
        ),
        mesh=mesh,
        scratch_types=[
            pltpu.VMEM((L,), jnp.int32),
            pltpu.VMEM((WPT, EMB, LANES), jnp.float32),
            pltpu.VMEM((EMB,), jnp.float32),
            pltpu.SemaphoreType.DMA,
        ],
        compiler_params=pltpu.CompilerParams(needs_layout_passes=False),
    )
    return f(ut, vt, words, h_path_pad)


def _tc_body(partials_ref, theta_ref, hcode_ref, out_ref):
    xw = jnp.sum(partials_ref[...], axis=0, keepdims=True)
    t = jnp.sum(theta_ref[...] * xw, axis=1, keepdims=True)
    z = jax.nn.sigmoid(t)
    hc = hcode_ref[...]
    row = lax.broadcasted_iota(jnp.int32, (PATH_PAD, 1), 0)
    loss = jnp.log(z) * hc + jnp.log(1.0 - z) * (1.0 - hc)
    loss = jnp.where(row < PATH, loss, 0.0)
    out_ref[...] = jnp.sum(loss, axis=0, keepdims=True)


def _tc_finish(partials, theta, h_code):
    hc = jnp.concatenate([h_code, jnp.zeros((PATH_PAD - PATH,), jnp.float32)])
    return pl.pallas_call(
        _tc_body,
        out_shape=jax.ShapeDtypeStruct((1, 1), jnp.float32),
    )(partials, theta, hc.reshape(PATH_PAD, 1))


def kernel(words, h_code, h_path, u_emb, v_emb):
    words = words.astype(jnp.int32)
    h_path = jnp.concatenate([h_path.astype(jnp.int32),
                              jnp.zeros((PATH_PAD - PATH,), jnp.int32)])
    partials, theta = _sc_gather(words, h_path, u_emb.T, v_emb.T)
    return _tc_finish(partials, theta, h_code)
